# baseline (speedup 1.0000x reference)
import functools

import jax
import jax.numpy as jnp
from jax import lax
from jax.experimental import pallas as pl
from jax.experimental.pallas import tpu as pltpu
from jax.experimental.pallas import tpu_sc as plsc

N = 10000
E = 320000
D = 128
ALPHA = 0.5
EPS_LSE = 1e-30
EPS_LN = 1e-5

BLK = 1000
NBLK = N // BLK

NC = 2
NS = 16
NW = NC * NS
EPW = E // NW
CH = 125
NCHUNK = EPW // CH
RPT = 624
TAIL = N - NS * RPT


def _proj_body(x_ref, w_ref, b_ref, h_ref, m_ref, p_ref):
    ph = pl.program_id(0)
    i = pl.program_id(1)

    @pl.when(ph == 0)
    def _():
        h = jnp.dot(x_ref[...], w_ref[...], preferred_element_type=jnp.float32)
        h = h + b_ref[...]
        h_ref[pl.ds(i * BLK, BLK), :] = h
        bm = jnp.max(h, axis=0, keepdims=True)

        @pl.when(i == 0)
        def _():
            m_ref[...] = bm

        @pl.when(i > 0)
        def _():
            m_ref[...] = jnp.maximum(m_ref[...], bm)

    @pl.when(ph == 1)
    def _():
        p_ref[...] = jnp.exp(h_ref[pl.ds(i * BLK, BLK), :] - m_ref[...])


def _layer_body(h_ref, a0_ref, a1_ref, m_ref, w_ref, b_ref, g_ref, be_ref,
                hn_ref, mo_ref, p_ref):
    ph = pl.program_id(0)
    i = pl.program_id(1)

    @pl.when(ph == 0)
    def _():
        h = h_ref[...]
        acc = a0_ref[...] + a1_ref[...]
        has = jnp.max(acc, axis=1, keepdims=True) > 0.0
        agg = jnp.where(has,
                        m_ref[...] + jnp.log(jnp.maximum(acc, EPS_LSE)), 0.0)
        z = (jnp.dot(h, w_ref[:D, :], preferred_element_type=jnp.float32)
             + jnp.dot(agg, w_ref[D:, :], preferred_element_type=jnp.float32)
             + b_ref[...])
        mu = jnp.mean(z, axis=1, keepdims=True)
        zc = z - mu
        var = jnp.mean(zc * zc, axis=1, keepdims=True)
        zn = zc * lax.rsqrt(var + EPS_LN) * g_ref[...] + be_ref[...]
        hn = ALPHA * h + (1.0 - ALPHA) * jnp.maximum(zn, 0.0)
        hn_ref[pl.ds(i * BLK, BLK), :] = hn
        bm = jnp.max(hn, axis=0, keepdims=True)

        @pl.when(i == 0)
        def _():
            mo_ref[...] = bm

        @pl.when(i > 0)
        def _():
            mo_ref[...] = jnp.maximum(mo_ref[...], bm)

    @pl.when(ph == 1)
    def _():
        p_ref[...] = jnp.exp(hn_ref[pl.ds(i * BLK, BLK), :] - mo_ref[...])


_in_spec = pl.BlockSpec((BLK, D), lambda p, i: (i * (1 - p), 0))
_w_spec = lambda r: pl.BlockSpec((r, D), lambda p, i: (0, 0))
_vec_spec = pl.BlockSpec((1, D), lambda p, i: (0, 0))
_resident_spec = pl.BlockSpec((N, D), lambda p, i: (0, 0))
_p_spec = pl.BlockSpec((BLK, D), lambda p, i: (i * p, 0))

_proj = pl.pallas_call(
    _proj_body,
    grid=(2, NBLK),
    in_specs=[_in_spec, _w_spec(D), _vec_spec],
    out_specs=[_resident_spec, _vec_spec, _p_spec],
    out_shape=[jax.ShapeDtypeStruct((N, D), jnp.float32),
               jax.ShapeDtypeStruct((1, D), jnp.float32),
               jax.ShapeDtypeStruct((N, D), jnp.float32)],
)

_layer = pl.pallas_call(
    _layer_body,
    grid=(2, NBLK),
    in_specs=[_in_spec, _in_spec, _in_spec, _vec_spec, _w_spec(2 * D),
              _vec_spec, _vec_spec, _vec_spec],
    out_specs=[_resident_spec, _vec_spec, _p_spec],
    out_shape=[jax.ShapeDtypeStruct((N, D), jnp.float32),
               jax.ShapeDtypeStruct((1, D), jnp.float32),
               jax.ShapeDtypeStruct((N, D), jnp.float32)],
)


def _scatter_body(p_hbm, src_hbm, dst_hbm, out_hbm,
                  sbuf0, sbuf1, sbuf2, dbuf0, dbuf1, dbuf2,
                  rows0, rows1, rows2, acc,
                  sg0, sg1, sg2, sd0, sd1, sd2, ss0, ss1, ss2,
                  sr0, sr1, sr2):
    cid = lax.axis_index("c")
    sid = lax.axis_index("s")
    wid = sid * NC + cid
    sbuf = (sbuf0, sbuf1, sbuf2)
    dbuf = (dbuf0, dbuf1, dbuf2)
    rows = (rows0, rows1, rows2)
    sg = (sg0, sg1, sg2)
    sd = (sd0, sd1, sd2)
    ss = (ss0, ss1, ss2)
    sr = (sr0, sr1, sr2)

    zv = jnp.zeros((16,), jnp.float32)

    def zfill(i, _):
        rows0[i // (D // 16), pl.ds((i % (D // 16)) * 16, 16)] = zv
        return 0

    lax.fori_loop(0, CH * (D // 16), zfill, 0)

    def zcopy(j, _):
        pltpu.sync_copy(rows0, acc.at[pl.ds(sid * RPT + j * CH, CH)])
        return 0

    lax.fori_loop(0, RPT // CH, zcopy, 0)

    zrem = RPT - (RPT // CH) * CH
    pltpu.sync_copy(rows0.at[pl.ds(0, zrem)],
                    acc.at[pl.ds(sid * RPT + (RPT // CH) * CH, zrem)])

    @pl.when(sid == NS - 1)
    def _():
        pltpu.sync_copy(rows0.at[pl.ds(0, TAIL)],
                        acc.at[pl.ds(NS * RPT, TAIL)])

    plsc.subcore_barrier()

    pltpu.async_copy(src_hbm.at[wid, 0], sbuf0, sr0)
    pltpu.async_copy(src_hbm.at[wid, 1], sbuf1, sr1)
    pltpu.async_copy(dst_hbm.at[wid, 0], dbuf0, sd0)
    pltpu.async_copy(dst_hbm.at[wid, 1], dbuf1, sd1)
    pltpu.make_async_copy(src_hbm.at[wid, 0], sbuf0, sr0).wait()
    pltpu.async_copy(p_hbm.at[sbuf0], rows0, sg0)
    pltpu.make_async_copy(src_hbm.at[wid, 1], sbuf1, sr1).wait()
    pltpu.async_copy(p_hbm.at[sbuf1], rows1, sg1)
    pltpu.async_copy(src_hbm.at[wid, 2], sbuf2, sr2)

    def stage(i, b):
        bp = (b + 2) % 3
        i = jnp.int32(i)
        pltpu.make_async_copy(p_hbm.at[sbuf[b]], rows[b], sg[b]).wait()
        pltpu.make_async_copy(dst_hbm.at[wid, i], dbuf[b], sd[b]).wait()
        pltpu.async_copy(rows[b], acc.at[dbuf[b]], ss[b], add=True)

        @pl.when(i >= 1)
        def _():
            pltpu.make_async_copy(rows[bp], acc.at[dbuf[bp]], ss[bp]).wait()

        @pl.when(i + 2 < NCHUNK)
        def _():
            pltpu.async_copy(dst_hbm.at[wid, i + 2], dbuf[bp], sd[bp])
            pltpu.make_async_copy(src_hbm.at[wid, i + 2], sbuf[bp],
                                  sr[bp]).wait()
            pltpu.async_copy(p_hbm.at[sbuf[bp]], rows[bp], sg[bp])

        @pl.when(i + 3 < NCHUNK)
        def _():
            pltpu.async_copy(src_hbm.at[wid, i + 3], sbuf[b], sr[b])

    def triple(j, _):
        stage(3 * j, 0)
        stage(3 * j + 1, 1)
        stage(3 * j + 2, 2)
        return 0

    nt = NCHUNK // 3
    lax.fori_loop(0, nt, triple, 0)
    for k in range(NCHUNK - 3 * nt):
        stage(3 * nt + k, k)
    lb = (NCHUNK - 1) % 3
    pltpu.make_async_copy(rows[lb], acc.at[dbuf[lb]], ss[lb]).wait()

    plsc.subcore_barrier()

    r0 = sid * RPT
    pltpu.sync_copy(acc.at[pl.ds(r0, RPT)], out_hbm.at[cid, pl.ds(r0, RPT)])

    @pl.when(sid == NS - 1)
    def _():
        pltpu.sync_copy(acc.at[pl.ds(NS * RPT, TAIL)],
                        out_hbm.at[cid, pl.ds(NS * RPT, TAIL)])


@functools.cache
def _make_scatter():
    return pl.kernel(
        _scatter_body,
        out_type=jax.ShapeDtypeStruct((NC, N, D), jnp.float32),
        mesh=plsc.VectorSubcoreMesh(core_axis_name="c", subcore_axis_name="s",
                                    num_cores=NC, num_subcores=NS),
        scratch_types=[
            pltpu.VMEM((CH,), jnp.int32),
            pltpu.VMEM((CH,), jnp.int32),
            pltpu.VMEM((CH,), jnp.int32),
            pltpu.VMEM((CH,), jnp.int32),
            pltpu.VMEM((CH,), jnp.int32),
            pltpu.VMEM((CH,), jnp.int32),
            pltpu.VMEM((CH, D), jnp.float32),
            pltpu.VMEM((CH, D), jnp.float32),
            pltpu.VMEM((CH, D), jnp.float32),
            pltpu.VMEM_SHARED((N, D), jnp.float32),
        ] + [pltpu.SemaphoreType.DMA] * 12,
    )


def kernel(x, edge_src, edge_dst, W_in, b_in, W1, b1, g1, be1,
           W2, b2, g2, be2, W3, b3, g3, be3):
    b_in = b_in.reshape(1, D)
    edge_src = edge_src.reshape(NW, NCHUNK, CH)
    edge_dst = edge_dst.reshape(NW, NCHUNK, CH)
    h, m, p = _proj(x, W_in, b_in)
    for (W, b, g, be) in ((W1, b1, g1, be1), (W2, b2, g2, be2),
                          (W3, b3, g3, be3)):
        parts = _make_scatter()(p, edge_src, edge_dst)
        h, m, p = _layer(h, parts[0], parts[1], m, W,
                         b.reshape(1, D), g.reshape(1, D), be.reshape(1, D))
    return h


# speedup vs baseline: 1.0005x; 1.0005x over previous
import functools

import jax
import jax.numpy as jnp
from jax import lax
from jax.experimental import pallas as pl
from jax.experimental.pallas import tpu as pltpu
from jax.experimental.pallas import tpu_sc as plsc

N = 10000
E = 320000
D = 128
ALPHA = 0.5
EPS_LSE = 1e-30
EPS_LN = 1e-5

BLK = 1000
NBLK = N // BLK

NC = 2
NS = 16
NW = NC * NS
EPW = E // NW
CH = 125
NCHUNK = EPW // CH
RPT = 624
TAIL = N - NS * RPT


def _proj_body(x_ref, w_ref, b_ref, h_ref, m_ref, p_ref):
    ph = pl.program_id(0)
    i = pl.program_id(1)

    @pl.when(ph == 0)
    def _():
        h = jnp.dot(x_ref[...], w_ref[...], preferred_element_type=jnp.float32)
        h = h + b_ref[...]
        h_ref[pl.ds(i * BLK, BLK), :] = h
        bm = jnp.max(h, axis=0, keepdims=True)

        @pl.when(i == 0)
        def _():
            m_ref[...] = bm

        @pl.when(i > 0)
        def _():
            m_ref[...] = jnp.maximum(m_ref[...], bm)

    @pl.when(ph == 1)
    def _():
        p_ref[...] = jnp.exp(h_ref[pl.ds(i * BLK, BLK), :] - m_ref[...])


def _layer_body(h_ref, a0_ref, a1_ref, m_ref, w_ref, b_ref, g_ref, be_ref,
                hn_ref, mo_ref, p_ref):
    ph = pl.program_id(0)
    i = pl.program_id(1)

    @pl.when(ph == 0)
    def _():
        h = h_ref[...]
        acc = a0_ref[...] + a1_ref[...]
        has = jnp.max(acc, axis=1, keepdims=True) > 0.0
        agg = jnp.where(has,
                        m_ref[...] + jnp.log(jnp.maximum(acc, EPS_LSE)), 0.0)
        z = (jnp.dot(h, w_ref[:D, :], preferred_element_type=jnp.float32)
             + jnp.dot(agg, w_ref[D:, :], preferred_element_type=jnp.float32)
             + b_ref[...])
        mu = jnp.mean(z, axis=1, keepdims=True)
        zc = z - mu
        var = jnp.mean(zc * zc, axis=1, keepdims=True)
        zn = zc * lax.rsqrt(var + EPS_LN) * g_ref[...] + be_ref[...]
        hn = ALPHA * h + (1.0 - ALPHA) * jnp.maximum(zn, 0.0)
        hn_ref[pl.ds(i * BLK, BLK), :] = hn
        bm = jnp.max(hn, axis=0, keepdims=True)

        @pl.when(i == 0)
        def _():
            mo_ref[...] = bm

        @pl.when(i > 0)
        def _():
            mo_ref[...] = jnp.maximum(mo_ref[...], bm)

    @pl.when(ph == 1)
    def _():
        p_ref[...] = jnp.exp(hn_ref[pl.ds(i * BLK, BLK), :] - mo_ref[...])


_in_spec = pl.BlockSpec((BLK, D), lambda p, i: (i * (1 - p), 0))
_w_spec = lambda r: pl.BlockSpec((r, D), lambda p, i: (0, 0))
_vec_spec = pl.BlockSpec((1, D), lambda p, i: (0, 0))
_resident_spec = pl.BlockSpec((N, D), lambda p, i: (0, 0))
_p_spec = pl.BlockSpec((BLK, D), lambda p, i: (i * p, 0))

_proj = pl.pallas_call(
    _proj_body,
    grid=(2, NBLK),
    in_specs=[_in_spec, _w_spec(D), _vec_spec],
    out_specs=[_resident_spec, _vec_spec, _p_spec],
    out_shape=[jax.ShapeDtypeStruct((N, D), jnp.float32),
               jax.ShapeDtypeStruct((1, D), jnp.float32),
               jax.ShapeDtypeStruct((N, D), jnp.float32)],
)

_layer = pl.pallas_call(
    _layer_body,
    grid=(2, NBLK),
    in_specs=[_in_spec, _in_spec, _in_spec, _vec_spec, _w_spec(2 * D),
              _vec_spec, _vec_spec, _vec_spec],
    out_specs=[_resident_spec, _vec_spec, _p_spec],
    out_shape=[jax.ShapeDtypeStruct((N, D), jnp.float32),
               jax.ShapeDtypeStruct((1, D), jnp.float32),
               jax.ShapeDtypeStruct((N, D), jnp.float32)],
)


CHA = 64
CHB = CH - CHA


def _gather2(p_hbm, sb, rw, sem):
    pltpu.async_copy(p_hbm.at[sb.at[pl.ds(0, CHA)]],
                     rw.at[pl.ds(0, CHA)], sem)
    pltpu.async_copy(p_hbm.at[sb.at[pl.ds(CHA, CHB)]],
                     rw.at[pl.ds(CHA, CHB)], sem)


def _scatter_body(p_hbm, src_hbm, dst_hbm, out_hbm,
                  sbuf0, sbuf1, sbuf2, dbuf0, dbuf1, dbuf2,
                  rows0, rows1, rows2, acc,
                  sg0, sg1, sg2, sd0, sd1, sd2, ss0, ss1, ss2,
                  sr0, sr1, sr2):
    cid = lax.axis_index("c")
    sid = lax.axis_index("s")
    wid = sid * NC + cid
    sbuf = (sbuf0, sbuf1, sbuf2)
    dbuf = (dbuf0, dbuf1, dbuf2)
    rows = (rows0, rows1, rows2)
    sg = (sg0, sg1, sg2)
    sd = (sd0, sd1, sd2)
    ss = (ss0, ss1, ss2)
    sr = (sr0, sr1, sr2)

    zv = jnp.zeros((16,), jnp.float32)

    def zfill(i, _):
        rows0[i // (D // 16), pl.ds((i % (D // 16)) * 16, 16)] = zv
        return 0

    lax.fori_loop(0, CH * (D // 16), zfill, 0)

    def zcopy(j, _):
        pltpu.sync_copy(rows0, acc.at[pl.ds(sid * RPT + j * CH, CH)])
        return 0

    lax.fori_loop(0, RPT // CH, zcopy, 0)

    zrem = RPT - (RPT // CH) * CH
    pltpu.sync_copy(rows0.at[pl.ds(0, zrem)],
                    acc.at[pl.ds(sid * RPT + (RPT // CH) * CH, zrem)])

    @pl.when(sid == NS - 1)
    def _():
        pltpu.sync_copy(rows0.at[pl.ds(0, TAIL)],
                        acc.at[pl.ds(NS * RPT, TAIL)])

    plsc.subcore_barrier()

    pltpu.async_copy(src_hbm.at[wid, 0], sbuf0, sr0)
    pltpu.async_copy(src_hbm.at[wid, 1], sbuf1, sr1)
    pltpu.async_copy(dst_hbm.at[wid, 0], dbuf0, sd0)
    pltpu.async_copy(dst_hbm.at[wid, 1], dbuf1, sd1)
    pltpu.make_async_copy(src_hbm.at[wid, 0], sbuf0, sr0).wait()
    _gather2(p_hbm, sbuf0, rows0, sg0)
    pltpu.make_async_copy(src_hbm.at[wid, 1], sbuf1, sr1).wait()
    _gather2(p_hbm, sbuf1, rows1, sg1)
    pltpu.async_copy(src_hbm.at[wid, 2], sbuf2, sr2)

    def stage(i, b):
        bp = (b + 2) % 3
        i = jnp.int32(i)
        pltpu.make_async_copy(p_hbm.at[sbuf[b].at[pl.ds(0, CHA)],
                              ], rows[b].at[pl.ds(0, CHA)], sg[b]).wait()
        pltpu.make_async_copy(p_hbm.at[sbuf[b].at[pl.ds(CHA, CHB)],
                              ], rows[b].at[pl.ds(CHA, CHB)], sg[b]).wait()
        pltpu.make_async_copy(dst_hbm.at[wid, i], dbuf[b], sd[b]).wait()
        pltpu.async_copy(rows[b], acc.at[dbuf[b]], ss[b], add=True)

        @pl.when(i >= 1)
        def _():
            pltpu.make_async_copy(rows[bp], acc.at[dbuf[bp]], ss[bp]).wait()

        @pl.when(i + 2 < NCHUNK)
        def _():
            pltpu.async_copy(dst_hbm.at[wid, i + 2], dbuf[bp], sd[bp])
            pltpu.make_async_copy(src_hbm.at[wid, i + 2], sbuf[bp],
                                  sr[bp]).wait()
            _gather2(p_hbm, sbuf[bp], rows[bp], sg[bp])

        @pl.when(i + 3 < NCHUNK)
        def _():
            pltpu.async_copy(src_hbm.at[wid, i + 3], sbuf[b], sr[b])

    def triple(j, _):
        stage(3 * j, 0)
        stage(3 * j + 1, 1)
        stage(3 * j + 2, 2)
        return 0

    nt = NCHUNK // 3
    lax.fori_loop(0, nt, triple, 0)
    for k in range(NCHUNK - 3 * nt):
        stage(3 * nt + k, k)
    lb = (NCHUNK - 1) % 3
    pltpu.make_async_copy(rows[lb], acc.at[dbuf[lb]], ss[lb]).wait()

    plsc.subcore_barrier()

    r0 = sid * RPT
    pltpu.sync_copy(acc.at[pl.ds(r0, RPT)], out_hbm.at[cid, pl.ds(r0, RPT)])

    @pl.when(sid == NS - 1)
    def _():
        pltpu.sync_copy(acc.at[pl.ds(NS * RPT, TAIL)],
                        out_hbm.at[cid, pl.ds(NS * RPT, TAIL)])


@functools.cache
def _make_scatter():
    return pl.kernel(
        _scatter_body,
        out_type=jax.ShapeDtypeStruct((NC, N, D), jnp.float32),
        mesh=plsc.VectorSubcoreMesh(core_axis_name="c", subcore_axis_name="s",
                                    num_cores=NC, num_subcores=NS),
        scratch_types=[
            pltpu.VMEM((CH,), jnp.int32),
            pltpu.VMEM((CH,), jnp.int32),
            pltpu.VMEM((CH,), jnp.int32),
            pltpu.VMEM((CH,), jnp.int32),
            pltpu.VMEM((CH,), jnp.int32),
            pltpu.VMEM((CH,), jnp.int32),
            pltpu.VMEM((CH, D), jnp.float32),
            pltpu.VMEM((CH, D), jnp.float32),
            pltpu.VMEM((CH, D), jnp.float32),
            pltpu.VMEM_SHARED((N, D), jnp.float32),
        ] + [pltpu.SemaphoreType.DMA] * 12,
    )


def kernel(x, edge_src, edge_dst, W_in, b_in, W1, b1, g1, be1,
           W2, b2, g2, be2, W3, b3, g3, be3):
    b_in = b_in.reshape(1, D)
    edge_src = edge_src.reshape(NW, NCHUNK, CH)
    edge_dst = edge_dst.reshape(NW, NCHUNK, CH)
    h, m, p = _proj(x, W_in, b_in)
    for (W, b, g, be) in ((W1, b1, g1, be1), (W2, b2, g2, be2),
                          (W3, b3, g3, be3)):
        parts = _make_scatter()(p, edge_src, edge_dst)
        h, m, p = _layer(h, parts[0], parts[1], m, W,
                         b.reshape(1, D), g.reshape(1, D), be.reshape(1, D))
    return h


# speedup vs baseline: 1.0397x; 1.0392x over previous
import functools

import jax
import jax.numpy as jnp
from jax import lax
from jax.experimental import pallas as pl
from jax.experimental.pallas import tpu as pltpu
from jax.experimental.pallas import tpu_sc as plsc

N = 10000
E = 320000
D = 128
ALPHA = 0.5
EPS_LSE = 1e-30
EPS_LN = 1e-5

BLK = 1000
NBLK = N // BLK

NC = 2
NS = 16
NW = NC * NS
EPW = E // NW
CH = 125
NCHUNK = EPW // CH
RPT = 624
TAIL = N - NS * RPT


def _proj_body(x_ref, w_ref, b_ref, h_ref, m_ref, p_ref):
    ph = pl.program_id(0)
    i = pl.program_id(1)

    @pl.when(ph == 0)
    def _():
        h = jnp.dot(x_ref[...], w_ref[...], preferred_element_type=jnp.float32)
        h = h + b_ref[...]
        h_ref[pl.ds(i * BLK, BLK), :] = h
        bm = jnp.max(h, axis=0, keepdims=True)

        @pl.when(i == 0)
        def _():
            m_ref[...] = bm

        @pl.when(i > 0)
        def _():
            m_ref[...] = jnp.maximum(m_ref[...], bm)

    @pl.when(ph == 1)
    def _():
        p_ref[...] = jnp.exp(h_ref[pl.ds(i * BLK, BLK), :] - m_ref[...])


def _layer_body(h_ref, a0_ref, a1_ref, m_ref, w_ref, b_ref, g_ref, be_ref,
                hn_ref, mo_ref, p_ref):
    ph = pl.program_id(0)
    i = pl.program_id(1)

    @pl.when(ph == 0)
    def _():
        h = h_ref[...]
        acc = a0_ref[...] + a1_ref[...]
        has = jnp.max(acc, axis=1, keepdims=True) > 0.0
        agg = jnp.where(has,
                        m_ref[...] + jnp.log(jnp.maximum(acc, EPS_LSE)), 0.0)
        z = (jnp.dot(h, w_ref[:D, :], preferred_element_type=jnp.float32)
             + jnp.dot(agg, w_ref[D:, :], preferred_element_type=jnp.float32)
             + b_ref[...])
        mu = jnp.mean(z, axis=1, keepdims=True)
        zc = z - mu
        var = jnp.mean(zc * zc, axis=1, keepdims=True)
        zn = zc * lax.rsqrt(var + EPS_LN) * g_ref[...] + be_ref[...]
        hn = ALPHA * h + (1.0 - ALPHA) * jnp.maximum(zn, 0.0)
        hn_ref[pl.ds(i * BLK, BLK), :] = hn
        bm = jnp.max(hn, axis=0, keepdims=True)

        @pl.when(i == 0)
        def _():
            mo_ref[...] = bm

        @pl.when(i > 0)
        def _():
            mo_ref[...] = jnp.maximum(mo_ref[...], bm)

    @pl.when(ph == 1)
    def _():
        p_ref[...] = jnp.exp(hn_ref[pl.ds(i * BLK, BLK), :] - mo_ref[...])


_in_spec = pl.BlockSpec((BLK, D), lambda p, i: (i * (1 - p), 0))
_w_spec = lambda r: pl.BlockSpec((r, D), lambda p, i: (0, 0))
_vec_spec = pl.BlockSpec((1, D), lambda p, i: (0, 0))
_resident_spec = pl.BlockSpec((N, D), lambda p, i: (0, 0))
_p_spec = pl.BlockSpec((BLK, D), lambda p, i: (i * p, 0))

_proj = pl.pallas_call(
    _proj_body,
    grid=(2, NBLK),
    in_specs=[_in_spec, _w_spec(D), _vec_spec],
    out_specs=[_resident_spec, _vec_spec, _p_spec],
    out_shape=[jax.ShapeDtypeStruct((N, D), jnp.float32),
               jax.ShapeDtypeStruct((1, D), jnp.float32),
               jax.ShapeDtypeStruct((N, D), jnp.float32)],
)

_layer = pl.pallas_call(
    _layer_body,
    grid=(2, NBLK),
    in_specs=[_in_spec, _in_spec, _in_spec, _vec_spec, _w_spec(2 * D),
              _vec_spec, _vec_spec, _vec_spec],
    out_specs=[_resident_spec, _vec_spec, _p_spec],
    out_shape=[jax.ShapeDtypeStruct((N, D), jnp.float32),
               jax.ShapeDtypeStruct((1, D), jnp.float32),
               jax.ShapeDtypeStruct((N, D), jnp.float32)],
)


def _last_body(h_ref, a0_ref, a1_ref, m_ref, w_ref, b_ref, g_ref, be_ref,
               hn_ref):
    h = h_ref[...]
    acc = a0_ref[...] + a1_ref[...]
    has = jnp.max(acc, axis=1, keepdims=True) > 0.0
    agg = jnp.where(has,
                    m_ref[...] + jnp.log(jnp.maximum(acc, EPS_LSE)), 0.0)
    z = (jnp.dot(h, w_ref[:D, :], preferred_element_type=jnp.float32)
         + jnp.dot(agg, w_ref[D:, :], preferred_element_type=jnp.float32)
         + b_ref[...])
    mu = jnp.mean(z, axis=1, keepdims=True)
    zc = z - mu
    var = jnp.mean(zc * zc, axis=1, keepdims=True)
    zn = zc * lax.rsqrt(var + EPS_LN) * g_ref[...] + be_ref[...]
    hn_ref[...] = ALPHA * h + (1.0 - ALPHA) * jnp.maximum(zn, 0.0)


_blk_spec = pl.BlockSpec((BLK, D), lambda i: (i, 0))

_last = pl.pallas_call(
    _last_body,
    grid=(NBLK,),
    in_specs=[_blk_spec, _blk_spec, _blk_spec,
              pl.BlockSpec((1, D), lambda i: (0, 0)),
              pl.BlockSpec((2 * D, D), lambda i: (0, 0)),
              pl.BlockSpec((1, D), lambda i: (0, 0)),
              pl.BlockSpec((1, D), lambda i: (0, 0)),
              pl.BlockSpec((1, D), lambda i: (0, 0))],
    out_specs=_blk_spec,
    out_shape=jax.ShapeDtypeStruct((N, D), jnp.float32),
)


def _scatter_body(p_hbm, src_hbm, dst_hbm, out_hbm,
                  sbuf0, sbuf1, sbuf2, dbuf0, dbuf1, dbuf2,
                  rows0, rows1, rows2, acc,
                  sg0, sg1, sg2, sd0, sd1, sd2, ss0, ss1, ss2,
                  sr0, sr1, sr2):
    cid = lax.axis_index("c")
    sid = lax.axis_index("s")
    wid = sid * NC + cid
    sbuf = (sbuf0, sbuf1, sbuf2)
    dbuf = (dbuf0, dbuf1, dbuf2)
    rows = (rows0, rows1, rows2)
    sg = (sg0, sg1, sg2)
    sd = (sd0, sd1, sd2)
    ss = (ss0, ss1, ss2)
    sr = (sr0, sr1, sr2)

    zv = jnp.zeros((16,), jnp.float32)

    def zfill(r, _):
        for c in range(D // 16):
            rows0[r, pl.ds(c * 16, 16)] = zv
        return 0

    lax.fori_loop(0, CH, zfill, 0)

    def zcopy(j, _):
        pltpu.sync_copy(rows0, acc.at[pl.ds(sid * RPT + j * CH, CH)])
        return 0

    lax.fori_loop(0, RPT // CH, zcopy, 0)

    zrem = RPT - (RPT // CH) * CH
    pltpu.sync_copy(rows0.at[pl.ds(0, zrem)],
                    acc.at[pl.ds(sid * RPT + (RPT // CH) * CH, zrem)])

    @pl.when(sid == NS - 1)
    def _():
        pltpu.sync_copy(rows0.at[pl.ds(0, TAIL)],
                        acc.at[pl.ds(NS * RPT, TAIL)])

    plsc.subcore_barrier()

    pltpu.async_copy(src_hbm.at[wid, 0], sbuf0, sr0)
    pltpu.async_copy(src_hbm.at[wid, 1], sbuf1, sr1)
    pltpu.async_copy(dst_hbm.at[wid, 0], dbuf0, sd0)
    pltpu.async_copy(dst_hbm.at[wid, 1], dbuf1, sd1)
    pltpu.make_async_copy(src_hbm.at[wid, 0], sbuf0, sr0).wait()
    pltpu.async_copy(p_hbm.at[sbuf0], rows0, sg0)
    pltpu.make_async_copy(src_hbm.at[wid, 1], sbuf1, sr1).wait()
    pltpu.async_copy(p_hbm.at[sbuf1], rows1, sg1)
    pltpu.async_copy(src_hbm.at[wid, 2], sbuf2, sr2)

    def stage(i, b):
        bp = (b + 2) % 3
        i = jnp.int32(i)
        pltpu.make_async_copy(p_hbm.at[sbuf[b]], rows[b], sg[b]).wait()
        pltpu.make_async_copy(dst_hbm.at[wid, i], dbuf[b], sd[b]).wait()
        pltpu.async_copy(rows[b], acc.at[dbuf[b]], ss[b], add=True)

        @pl.when(i >= 1)
        def _():
            pltpu.make_async_copy(rows[bp], acc.at[dbuf[bp]], ss[bp]).wait()

        @pl.when(i + 2 < NCHUNK)
        def _():
            pltpu.async_copy(dst_hbm.at[wid, i + 2], dbuf[bp], sd[bp])
            pltpu.make_async_copy(src_hbm.at[wid, i + 2], sbuf[bp],
                                  sr[bp]).wait()
            pltpu.async_copy(p_hbm.at[sbuf[bp]], rows[bp], sg[bp])

        @pl.when(i + 3 < NCHUNK)
        def _():
            pltpu.async_copy(src_hbm.at[wid, i + 3], sbuf[b], sr[b])

    def triple(j, _):
        stage(3 * j, 0)
        stage(3 * j + 1, 1)
        stage(3 * j + 2, 2)
        return 0

    nt = NCHUNK // 3
    lax.fori_loop(0, nt, triple, 0)
    for k in range(NCHUNK - 3 * nt):
        stage(3 * nt + k, k)
    lb = (NCHUNK - 1) % 3
    pltpu.make_async_copy(rows[lb], acc.at[dbuf[lb]], ss[lb]).wait()

    plsc.subcore_barrier()

    r0 = sid * RPT
    pltpu.sync_copy(acc.at[pl.ds(r0, RPT)], out_hbm.at[cid, pl.ds(r0, RPT)])

    @pl.when(sid == NS - 1)
    def _():
        pltpu.sync_copy(acc.at[pl.ds(NS * RPT, TAIL)],
                        out_hbm.at[cid, pl.ds(NS * RPT, TAIL)])


@functools.cache
def _make_scatter():
    return pl.kernel(
        _scatter_body,
        out_type=jax.ShapeDtypeStruct((NC, N, D), jnp.float32),
        mesh=plsc.VectorSubcoreMesh(core_axis_name="c", subcore_axis_name="s",
                                    num_cores=NC, num_subcores=NS),
        scratch_types=[
            pltpu.VMEM((CH,), jnp.int32),
            pltpu.VMEM((CH,), jnp.int32),
            pltpu.VMEM((CH,), jnp.int32),
            pltpu.VMEM((CH,), jnp.int32),
            pltpu.VMEM((CH,), jnp.int32),
            pltpu.VMEM((CH,), jnp.int32),
            pltpu.VMEM((CH, D), jnp.float32),
            pltpu.VMEM((CH, D), jnp.float32),
            pltpu.VMEM((CH, D), jnp.float32),
            pltpu.VMEM_SHARED((N, D), jnp.float32),
        ] + [pltpu.SemaphoreType.DMA] * 12,
    )


def kernel(x, edge_src, edge_dst, W_in, b_in, W1, b1, g1, be1,
           W2, b2, g2, be2, W3, b3, g3, be3):
    b_in = b_in.reshape(1, D)
    edge_src = edge_src.reshape(NW, NCHUNK, CH)
    edge_dst = edge_dst.reshape(NW, NCHUNK, CH)
    h, m, p = _proj(x, W_in, b_in)
    for (W, b, g, be) in ((W1, b1, g1, be1), (W2, b2, g2, be2)):
        parts = _make_scatter()(p, edge_src, edge_dst)
        h, m, p = _layer(h, parts[0], parts[1], m, W,
                         b.reshape(1, D), g.reshape(1, D), be.reshape(1, D))
    parts = _make_scatter()(p, edge_src, edge_dst)
    return _last(h, parts[0], parts[1], m, W3,
                 b3.reshape(1, D), g3.reshape(1, D), be3.reshape(1, D))


# speedup vs baseline: 1.0573x; 1.0170x over previous
import functools

import jax
import jax.numpy as jnp
from jax import lax
from jax.experimental import pallas as pl
from jax.experimental.pallas import tpu as pltpu
from jax.experimental.pallas import tpu_sc as plsc

N = 10000
E = 320000
D = 128
ALPHA = 0.5
EPS_LSE = 1e-30
EPS_LN = 1e-5

BLK = 1000
NBLK = N // BLK

NC = 2
NS = 16
NW = NC * NS
EPW = E // NW
CH = 125
NCHUNK = EPW // CH
RPT = 624
TAIL = N - NS * RPT


def _proj_body(x_ref, w_ref, b_ref, h_ref, m_ref, p_ref):
    ph = pl.program_id(0)
    i = pl.program_id(1)

    @pl.when(ph == 0)
    def _():
        h = jnp.dot(x_ref[...], w_ref[...], preferred_element_type=jnp.float32)
        h = h + b_ref[...]
        h_ref[pl.ds(i * BLK, BLK), :] = h
        bm = jnp.max(h, axis=0, keepdims=True)

        @pl.when(i == 0)
        def _():
            m_ref[...] = bm

        @pl.when(i > 0)
        def _():
            m_ref[...] = jnp.maximum(m_ref[...], bm)

    @pl.when(ph == 1)
    def _():
        p_ref[...] = jnp.exp(h_ref[pl.ds(i * BLK, BLK), :] - m_ref[...])


def _layer_body(h_ref, a0_ref, a1_ref, m_ref, w_ref, b_ref, g_ref, be_ref,
                hn_ref, mo_ref, p_ref):
    ph = pl.program_id(0)
    i = pl.program_id(1)

    @pl.when(ph == 0)
    def _():
        h = h_ref[...]
        acc = a0_ref[...] + a1_ref[...]
        has = jnp.max(acc, axis=1, keepdims=True) > 0.0
        agg = jnp.where(has,
                        m_ref[...] + jnp.log(jnp.maximum(acc, EPS_LSE)), 0.0)
        z = (jnp.dot(h, w_ref[:D, :], preferred_element_type=jnp.float32)
             + jnp.dot(agg, w_ref[D:, :], preferred_element_type=jnp.float32)
             + b_ref[...])
        mu = jnp.mean(z, axis=1, keepdims=True)
        zc = z - mu
        var = jnp.mean(zc * zc, axis=1, keepdims=True)
        zn = zc * lax.rsqrt(var + EPS_LN) * g_ref[...] + be_ref[...]
        hn = ALPHA * h + (1.0 - ALPHA) * jnp.maximum(zn, 0.0)
        hn_ref[pl.ds(i * BLK, BLK), :] = hn
        bm = jnp.max(hn, axis=0, keepdims=True)

        @pl.when(i == 0)
        def _():
            mo_ref[...] = bm

        @pl.when(i > 0)
        def _():
            mo_ref[...] = jnp.maximum(mo_ref[...], bm)

    @pl.when(ph == 1)
    def _():
        p_ref[...] = jnp.exp(hn_ref[pl.ds(i * BLK, BLK), :] - mo_ref[...])


_in_spec = pl.BlockSpec((BLK, D), lambda p, i: (i * (1 - p), 0))
_w_spec = lambda r: pl.BlockSpec((r, D), lambda p, i: (0, 0))
_vec_spec = pl.BlockSpec((1, D), lambda p, i: (0, 0))
_resident_spec = pl.BlockSpec((N, D), lambda p, i: (0, 0))
_p_spec = pl.BlockSpec((BLK, D), lambda p, i: (i * p, 0))

_proj = pl.pallas_call(
    _proj_body,
    grid=(2, NBLK),
    in_specs=[_in_spec, _w_spec(D), _vec_spec],
    out_specs=[_resident_spec, _vec_spec, _p_spec],
    out_shape=[jax.ShapeDtypeStruct((N, D), jnp.float32),
               jax.ShapeDtypeStruct((1, D), jnp.float32),
               jax.ShapeDtypeStruct((N, D), jnp.float32)],
)

_layer = pl.pallas_call(
    _layer_body,
    grid=(2, NBLK),
    in_specs=[_in_spec, _in_spec, _in_spec, _vec_spec, _w_spec(2 * D),
              _vec_spec, _vec_spec, _vec_spec],
    out_specs=[_resident_spec, _vec_spec, _p_spec],
    out_shape=[jax.ShapeDtypeStruct((N, D), jnp.float32),
               jax.ShapeDtypeStruct((1, D), jnp.float32),
               jax.ShapeDtypeStruct((N, D), jnp.float32)],
)


def _last_body(h_ref, a0_ref, a1_ref, m_ref, w_ref, b_ref, g_ref, be_ref,
               hn_ref):
    h = h_ref[...]
    acc = a0_ref[...] + a1_ref[...]
    has = jnp.max(acc, axis=1, keepdims=True) > 0.0
    agg = jnp.where(has,
                    m_ref[...] + jnp.log(jnp.maximum(acc, EPS_LSE)), 0.0)
    z = (jnp.dot(h, w_ref[:D, :], preferred_element_type=jnp.float32)
         + jnp.dot(agg, w_ref[D:, :], preferred_element_type=jnp.float32)
         + b_ref[...])
    mu = jnp.mean(z, axis=1, keepdims=True)
    zc = z - mu
    var = jnp.mean(zc * zc, axis=1, keepdims=True)
    zn = zc * lax.rsqrt(var + EPS_LN) * g_ref[...] + be_ref[...]
    hn_ref[...] = ALPHA * h + (1.0 - ALPHA) * jnp.maximum(zn, 0.0)


_blk_spec = pl.BlockSpec((BLK, D), lambda i: (i, 0))

_last = pl.pallas_call(
    _last_body,
    grid=(NBLK,),
    in_specs=[_blk_spec, _blk_spec, _blk_spec,
              pl.BlockSpec((1, D), lambda i: (0, 0)),
              pl.BlockSpec((2 * D, D), lambda i: (0, 0)),
              pl.BlockSpec((1, D), lambda i: (0, 0)),
              pl.BlockSpec((1, D), lambda i: (0, 0)),
              pl.BlockSpec((1, D), lambda i: (0, 0))],
    out_specs=_blk_spec,
    out_shape=jax.ShapeDtypeStruct((N, D), jnp.float32),
)


def _scatter_body(p_hbm, src_hbm, dst_hbm, out_hbm,
                  sbuf0, sbuf1, sbuf2, dbuf0, dbuf1, dbuf2,
                  rows0, rows1, rows2, acc,
                  sg0, sg1, sg2, sd0, sd1, sd2, ss0, ss1, ss2,
                  sr0, sr1, sr2):
    cid = lax.axis_index("c")
    sid = lax.axis_index("s")
    wid = sid * NC + cid
    sbuf = (sbuf0, sbuf1, sbuf2)
    dbuf = (dbuf0, dbuf1, dbuf2)
    rows = (rows0, rows1, rows2)
    sg = (sg0, sg1, sg2)
    sd = (sd0, sd1, sd2)
    ss = (ss0, ss1, ss2)
    sr = (sr0, sr1, sr2)

    pltpu.async_copy(src_hbm.at[wid, 0], sbuf0, sr0)
    pltpu.async_copy(src_hbm.at[wid, 1], sbuf1, sr1)
    pltpu.async_copy(dst_hbm.at[wid, 0], dbuf0, sd0)
    pltpu.async_copy(dst_hbm.at[wid, 1], dbuf1, sd1)
    pltpu.make_async_copy(src_hbm.at[wid, 0], sbuf0, sr0).wait()
    pltpu.async_copy(p_hbm.at[sbuf0], rows0, sg0)
    pltpu.make_async_copy(src_hbm.at[wid, 1], sbuf1, sr1).wait()
    pltpu.async_copy(p_hbm.at[sbuf1], rows1, sg1)
    pltpu.async_copy(src_hbm.at[wid, 2], sbuf2, sr2)

    zv = jnp.zeros((16,), jnp.float32)

    def zfill(r, _):
        for c in range(D // 16):
            rows2[r, pl.ds(c * 16, 16)] = zv
        return 0

    lax.fori_loop(0, CH, zfill, 0)

    def zcopy(j, _):
        pltpu.async_copy(rows2, acc.at[pl.ds(sid * RPT + j * CH, CH)], ss0)
        return 0

    lax.fori_loop(0, RPT // CH, zcopy, 0)

    zrem = RPT - (RPT // CH) * CH
    pltpu.async_copy(rows2.at[pl.ds(0, zrem)],
                     acc.at[pl.ds(sid * RPT + (RPT // CH) * CH, zrem)], ss0)

    @pl.when(sid == NS - 1)
    def _():
        pltpu.async_copy(rows2.at[pl.ds(0, TAIL)],
                         acc.at[pl.ds(NS * RPT, TAIL)], ss0)

    def zdrain(j, _):
        pltpu.make_async_copy(rows2, acc.at[pl.ds(sid * RPT + j * CH, CH)],
                              ss0).wait()
        return 0

    lax.fori_loop(0, RPT // CH, zdrain, 0)
    pltpu.make_async_copy(rows2.at[pl.ds(0, zrem)],
                          acc.at[pl.ds(sid * RPT, zrem)], ss0).wait()

    @pl.when(sid == NS - 1)
    def _():
        pltpu.make_async_copy(rows2.at[pl.ds(0, TAIL)],
                              acc.at[pl.ds(NS * RPT, TAIL)], ss0).wait()

    plsc.subcore_barrier()

    def stage(i, b):
        bp = (b + 2) % 3
        i = jnp.int32(i)
        pltpu.make_async_copy(p_hbm.at[sbuf[b]], rows[b], sg[b]).wait()
        pltpu.make_async_copy(dst_hbm.at[wid, i], dbuf[b], sd[b]).wait()
        pltpu.async_copy(rows[b], acc.at[dbuf[b]], ss[b], add=True)

        @pl.when(i >= 1)
        def _():
            pltpu.make_async_copy(rows[bp], acc.at[dbuf[bp]], ss[bp]).wait()

        @pl.when(i + 2 < NCHUNK)
        def _():
            pltpu.async_copy(dst_hbm.at[wid, i + 2], dbuf[bp], sd[bp])
            pltpu.make_async_copy(src_hbm.at[wid, i + 2], sbuf[bp],
                                  sr[bp]).wait()
            pltpu.async_copy(p_hbm.at[sbuf[bp]], rows[bp], sg[bp])

        @pl.when(i + 3 < NCHUNK)
        def _():
            pltpu.async_copy(src_hbm.at[wid, i + 3], sbuf[b], sr[b])

    def triple(j, _):
        stage(3 * j, 0)
        stage(3 * j + 1, 1)
        stage(3 * j + 2, 2)
        return 0

    nt = NCHUNK // 3
    lax.fori_loop(0, nt, triple, 0)
    for k in range(NCHUNK - 3 * nt):
        stage(3 * nt + k, k)
    lb = (NCHUNK - 1) % 3
    pltpu.make_async_copy(rows[lb], acc.at[dbuf[lb]], ss[lb]).wait()

    plsc.subcore_barrier()

    r0 = sid * RPT
    pltpu.sync_copy(acc.at[pl.ds(r0, RPT)], out_hbm.at[cid, pl.ds(r0, RPT)])

    @pl.when(sid == NS - 1)
    def _():
        pltpu.sync_copy(acc.at[pl.ds(NS * RPT, TAIL)],
                        out_hbm.at[cid, pl.ds(NS * RPT, TAIL)])


@functools.cache
def _make_scatter():
    return pl.kernel(
        _scatter_body,
        out_type=jax.ShapeDtypeStruct((NC, N, D), jnp.float32),
        mesh=plsc.VectorSubcoreMesh(core_axis_name="c", subcore_axis_name="s",
                                    num_cores=NC, num_subcores=NS),
        scratch_types=[
            pltpu.VMEM((CH,), jnp.int32),
            pltpu.VMEM((CH,), jnp.int32),
            pltpu.VMEM((CH,), jnp.int32),
            pltpu.VMEM((CH,), jnp.int32),
            pltpu.VMEM((CH,), jnp.int32),
            pltpu.VMEM((CH,), jnp.int32),
            pltpu.VMEM((CH, D), jnp.float32),
            pltpu.VMEM((CH, D), jnp.float32),
            pltpu.VMEM((CH, D), jnp.float32),
            pltpu.VMEM_SHARED((N, D), jnp.float32),
        ] + [pltpu.SemaphoreType.DMA] * 12,
    )


def kernel(x, edge_src, edge_dst, W_in, b_in, W1, b1, g1, be1,
           W2, b2, g2, be2, W3, b3, g3, be3):
    b_in = b_in.reshape(1, D)
    edge_src = edge_src.reshape(NW, NCHUNK, CH)
    edge_dst = edge_dst.reshape(NW, NCHUNK, CH)
    h, m, p = _proj(x, W_in, b_in)
    for (W, b, g, be) in ((W1, b1, g1, be1), (W2, b2, g2, be2)):
        parts = _make_scatter()(p, edge_src, edge_dst)
        h, m, p = _layer(h, parts[0], parts[1], m, W,
                         b.reshape(1, D), g.reshape(1, D), be.reshape(1, D))
    parts = _make_scatter()(p, edge_src, edge_dst)
    return _last(h, parts[0], parts[1], m, W3,
                 b3.reshape(1, D), g3.reshape(1, D), be3.reshape(1, D))


# speedup vs baseline: 1.0946x; 1.0352x over previous
import functools

import jax
import jax.numpy as jnp
from jax import lax
from jax.experimental import pallas as pl
from jax.experimental.pallas import tpu as pltpu
from jax.experimental.pallas import tpu_sc as plsc

N = 10000
E = 320000
D = 128
ALPHA = 0.5
EPS_LSE = 1e-30
EPS_LN = 1e-5

BLK = 2000
NBLK = N // BLK

NC = 2
NS = 16
NW = NC * NS
EPW = E // NW
CH = 125
NCHUNK = EPW // CH
RPT = 624
TAIL = N - NS * RPT


def _proj_body(x_ref, w_ref, b_ref, h_ref, m_ref, p_ref):
    ph = pl.program_id(0)
    i = pl.program_id(1)

    @pl.when(ph == 0)
    def _():
        h = jnp.dot(x_ref[...], w_ref[...], preferred_element_type=jnp.float32)
        h = h + b_ref[...]
        h_ref[pl.ds(i * BLK, BLK), :] = h
        bm = jnp.max(h, axis=0, keepdims=True)

        @pl.when(i == 0)
        def _():
            m_ref[...] = bm

        @pl.when(i > 0)
        def _():
            m_ref[...] = jnp.maximum(m_ref[...], bm)

    @pl.when(ph == 1)
    def _():
        p_ref[...] = jnp.exp(h_ref[pl.ds(i * BLK, BLK), :] - m_ref[...])


def _layer_body(h_ref, a0_ref, a1_ref, m_ref, w_ref, b_ref, g_ref, be_ref,
                hn_ref, mo_ref, p_ref):
    ph = pl.program_id(0)
    i = pl.program_id(1)

    @pl.when(ph == 0)
    def _():
        h = h_ref[...]
        acc = a0_ref[...] + a1_ref[...]
        has = jnp.max(acc, axis=1, keepdims=True) > 0.0
        agg = jnp.where(has,
                        m_ref[...] + jnp.log(jnp.maximum(acc, EPS_LSE)), 0.0)
        z = (jnp.dot(h, w_ref[:D, :], preferred_element_type=jnp.float32)
             + jnp.dot(agg, w_ref[D:, :], preferred_element_type=jnp.float32)
             + b_ref[...])
        mu = jnp.mean(z, axis=1, keepdims=True)
        zc = z - mu
        var = jnp.mean(zc * zc, axis=1, keepdims=True)
        zn = zc * lax.rsqrt(var + EPS_LN) * g_ref[...] + be_ref[...]
        hn = ALPHA * h + (1.0 - ALPHA) * jnp.maximum(zn, 0.0)
        hn_ref[pl.ds(i * BLK, BLK), :] = hn
        bm = jnp.max(hn, axis=0, keepdims=True)

        @pl.when(i == 0)
        def _():
            mo_ref[...] = bm

        @pl.when(i > 0)
        def _():
            mo_ref[...] = jnp.maximum(mo_ref[...], bm)

    @pl.when(ph == 1)
    def _():
        p_ref[...] = jnp.exp(hn_ref[pl.ds(i * BLK, BLK), :] - mo_ref[...])


_in_spec = pl.BlockSpec((BLK, D), lambda p, i: (i * (1 - p), 0))
_w_spec = lambda r: pl.BlockSpec((r, D), lambda p, i: (0, 0))
_vec_spec = pl.BlockSpec((1, D), lambda p, i: (0, 0))
_resident_spec = pl.BlockSpec((N, D), lambda p, i: (0, 0))
_p_spec = pl.BlockSpec((BLK, D), lambda p, i: (i * p, 0))

_proj = pl.pallas_call(
    _proj_body,
    grid=(2, NBLK),
    in_specs=[_in_spec, _w_spec(D), _vec_spec],
    out_specs=[_resident_spec, _vec_spec, _p_spec],
    out_shape=[jax.ShapeDtypeStruct((N, D), jnp.float32),
               jax.ShapeDtypeStruct((1, D), jnp.float32),
               jax.ShapeDtypeStruct((N, D), jnp.float32)],
)

_layer = pl.pallas_call(
    _layer_body,
    grid=(2, NBLK),
    in_specs=[_in_spec, _in_spec, _in_spec, _vec_spec, _w_spec(2 * D),
              _vec_spec, _vec_spec, _vec_spec],
    out_specs=[_resident_spec, _vec_spec, _p_spec],
    out_shape=[jax.ShapeDtypeStruct((N, D), jnp.float32),
               jax.ShapeDtypeStruct((1, D), jnp.float32),
               jax.ShapeDtypeStruct((N, D), jnp.float32)],
)


def _last_body(h_ref, a0_ref, a1_ref, m_ref, w_ref, b_ref, g_ref, be_ref,
               hn_ref):
    h = h_ref[...]
    acc = a0_ref[...] + a1_ref[...]
    has = jnp.max(acc, axis=1, keepdims=True) > 0.0
    agg = jnp.where(has,
                    m_ref[...] + jnp.log(jnp.maximum(acc, EPS_LSE)), 0.0)
    z = (jnp.dot(h, w_ref[:D, :], preferred_element_type=jnp.float32)
         + jnp.dot(agg, w_ref[D:, :], preferred_element_type=jnp.float32)
         + b_ref[...])
    mu = jnp.mean(z, axis=1, keepdims=True)
    zc = z - mu
    var = jnp.mean(zc * zc, axis=1, keepdims=True)
    zn = zc * lax.rsqrt(var + EPS_LN) * g_ref[...] + be_ref[...]
    hn_ref[...] = ALPHA * h + (1.0 - ALPHA) * jnp.maximum(zn, 0.0)


_blk_spec = pl.BlockSpec((BLK, D), lambda i: (i, 0))

_last = pl.pallas_call(
    _last_body,
    grid=(NBLK,),
    in_specs=[_blk_spec, _blk_spec, _blk_spec,
              pl.BlockSpec((1, D), lambda i: (0, 0)),
              pl.BlockSpec((2 * D, D), lambda i: (0, 0)),
              pl.BlockSpec((1, D), lambda i: (0, 0)),
              pl.BlockSpec((1, D), lambda i: (0, 0)),
              pl.BlockSpec((1, D), lambda i: (0, 0))],
    out_specs=_blk_spec,
    out_shape=jax.ShapeDtypeStruct((N, D), jnp.float32),
)


def _scatter_body(p_hbm, src_hbm, dst_hbm, out_hbm,
                  sbuf0, sbuf1, sbuf2, dbuf0, dbuf1, dbuf2,
                  rows0, rows1, rows2, acc,
                  sg0, sg1, sg2, sd0, sd1, sd2, ss0, ss1, ss2,
                  sr0, sr1, sr2):
    cid = lax.axis_index("c")
    sid = lax.axis_index("s")
    wid = sid * NC + cid
    sbuf = (sbuf0, sbuf1, sbuf2)
    dbuf = (dbuf0, dbuf1, dbuf2)
    rows = (rows0, rows1, rows2)
    sg = (sg0, sg1, sg2)
    sd = (sd0, sd1, sd2)
    ss = (ss0, ss1, ss2)
    sr = (sr0, sr1, sr2)

    pltpu.async_copy(src_hbm.at[wid, 0], sbuf0, sr0)
    pltpu.async_copy(src_hbm.at[wid, 1], sbuf1, sr1)
    pltpu.async_copy(dst_hbm.at[wid, 0], dbuf0, sd0)
    pltpu.async_copy(dst_hbm.at[wid, 1], dbuf1, sd1)
    pltpu.make_async_copy(src_hbm.at[wid, 0], sbuf0, sr0).wait()
    pltpu.async_copy(p_hbm.at[sbuf0], rows0, sg0)
    pltpu.make_async_copy(src_hbm.at[wid, 1], sbuf1, sr1).wait()
    pltpu.async_copy(p_hbm.at[sbuf1], rows1, sg1)
    pltpu.async_copy(src_hbm.at[wid, 2], sbuf2, sr2)

    zv = jnp.zeros((16,), jnp.float32)

    def zfill(r, _):
        for c in range(D // 16):
            rows2[r, pl.ds(c * 16, 16)] = zv
        return 0

    lax.fori_loop(0, CH, zfill, 0)

    def zcopy(j, _):
        pltpu.async_copy(rows2, acc.at[pl.ds(sid * RPT + j * CH, CH)], ss0)
        return 0

    lax.fori_loop(0, RPT // CH, zcopy, 0)

    zrem = RPT - (RPT // CH) * CH
    pltpu.async_copy(rows2.at[pl.ds(0, zrem)],
                     acc.at[pl.ds(sid * RPT + (RPT // CH) * CH, zrem)], ss0)

    @pl.when(sid == NS - 1)
    def _():
        pltpu.async_copy(rows2.at[pl.ds(0, TAIL)],
                         acc.at[pl.ds(NS * RPT, TAIL)], ss0)

    def zdrain(j, _):
        pltpu.make_async_copy(rows2, acc.at[pl.ds(sid * RPT + j * CH, CH)],
                              ss0).wait()
        return 0

    lax.fori_loop(0, RPT // CH, zdrain, 0)
    pltpu.make_async_copy(rows2.at[pl.ds(0, zrem)],
                          acc.at[pl.ds(sid * RPT, zrem)], ss0).wait()

    @pl.when(sid == NS - 1)
    def _():
        pltpu.make_async_copy(rows2.at[pl.ds(0, TAIL)],
                              acc.at[pl.ds(NS * RPT, TAIL)], ss0).wait()

    plsc.subcore_barrier()

    def stage(i, b):
        bp = (b + 2) % 3
        i = jnp.int32(i)
        pltpu.make_async_copy(p_hbm.at[sbuf[b]], rows[b], sg[b]).wait()
        pltpu.make_async_copy(dst_hbm.at[wid, i], dbuf[b], sd[b]).wait()
        pltpu.async_copy(rows[b], acc.at[dbuf[b]], ss[b], add=True)

        @pl.when(i >= 1)
        def _():
            pltpu.make_async_copy(rows[bp], acc.at[dbuf[bp]], ss[bp]).wait()

        @pl.when(i + 2 < NCHUNK)
        def _():
            pltpu.async_copy(dst_hbm.at[wid, i + 2], dbuf[bp], sd[bp])
            pltpu.make_async_copy(src_hbm.at[wid, i + 2], sbuf[bp],
                                  sr[bp]).wait()
            pltpu.async_copy(p_hbm.at[sbuf[bp]], rows[bp], sg[bp])

        @pl.when(i + 3 < NCHUNK)
        def _():
            pltpu.async_copy(src_hbm.at[wid, i + 3], sbuf[b], sr[b])

    def triple(j, _):
        stage(3 * j, 0)
        stage(3 * j + 1, 1)
        stage(3 * j + 2, 2)
        return 0

    nt = NCHUNK // 3
    lax.fori_loop(0, nt, triple, 0)
    for k in range(NCHUNK - 3 * nt):
        stage(3 * nt + k, k)
    lb = (NCHUNK - 1) % 3
    pltpu.make_async_copy(rows[lb], acc.at[dbuf[lb]], ss[lb]).wait()

    plsc.subcore_barrier()

    r0 = sid * RPT
    pltpu.sync_copy(acc.at[pl.ds(r0, RPT)], out_hbm.at[cid, pl.ds(r0, RPT)])

    @pl.when(sid == NS - 1)
    def _():
        pltpu.sync_copy(acc.at[pl.ds(NS * RPT, TAIL)],
                        out_hbm.at[cid, pl.ds(NS * RPT, TAIL)])


@functools.cache
def _make_scatter():
    return pl.kernel(
        _scatter_body,
        out_type=jax.ShapeDtypeStruct((NC, N, D), jnp.float32),
        mesh=plsc.VectorSubcoreMesh(core_axis_name="c", subcore_axis_name="s",
                                    num_cores=NC, num_subcores=NS),
        scratch_types=[
            pltpu.VMEM((CH,), jnp.int32),
            pltpu.VMEM((CH,), jnp.int32),
            pltpu.VMEM((CH,), jnp.int32),
            pltpu.VMEM((CH,), jnp.int32),
            pltpu.VMEM((CH,), jnp.int32),
            pltpu.VMEM((CH,), jnp.int32),
            pltpu.VMEM((CH, D), jnp.float32),
            pltpu.VMEM((CH, D), jnp.float32),
            pltpu.VMEM((CH, D), jnp.float32),
            pltpu.VMEM_SHARED((N, D), jnp.float32),
        ] + [pltpu.SemaphoreType.DMA] * 12,
    )


def kernel(x, edge_src, edge_dst, W_in, b_in, W1, b1, g1, be1,
           W2, b2, g2, be2, W3, b3, g3, be3):
    b_in = b_in.reshape(1, D)
    edge_src = edge_src.reshape(NW, NCHUNK, CH)
    edge_dst = edge_dst.reshape(NW, NCHUNK, CH)
    h, m, p = _proj(x, W_in, b_in)
    for (W, b, g, be) in ((W1, b1, g1, be1), (W2, b2, g2, be2)):
        parts = _make_scatter()(p, edge_src, edge_dst)
        h, m, p = _layer(h, parts[0], parts[1], m, W,
                         b.reshape(1, D), g.reshape(1, D), be.reshape(1, D))
    parts = _make_scatter()(p, edge_src, edge_dst)
    return _last(h, parts[0], parts[1], m, W3,
                 b3.reshape(1, D), g3.reshape(1, D), be3.reshape(1, D))


# speedup vs baseline: 1.1054x; 1.0099x over previous
import functools

import jax
import jax.numpy as jnp
from jax import lax
from jax.experimental import pallas as pl
from jax.experimental.pallas import tpu as pltpu
from jax.experimental.pallas import tpu_sc as plsc

N = 10000
E = 320000
D = 128
ALPHA = 0.5
EPS_LSE = 1e-30
EPS_LN = 1e-5

BLK = 5000
NBLK = N // BLK

NC = 2
NS = 16
NW = NC * NS
EPW = E // NW
CH = 125
NCHUNK = EPW // CH
RPT = 624
TAIL = N - NS * RPT


def _proj_body(x_ref, w_ref, b_ref, h_ref, m_ref, p_ref):
    ph = pl.program_id(0)
    i = pl.program_id(1)

    @pl.when(ph == 0)
    def _():
        h = jnp.dot(x_ref[...], w_ref[...], preferred_element_type=jnp.float32)
        h = h + b_ref[...]
        h_ref[pl.ds(i * BLK, BLK), :] = h
        bm = jnp.max(h, axis=0, keepdims=True)

        @pl.when(i == 0)
        def _():
            m_ref[...] = bm

        @pl.when(i > 0)
        def _():
            m_ref[...] = jnp.maximum(m_ref[...], bm)

    @pl.when(ph == 1)
    def _():
        p_ref[...] = jnp.exp(h_ref[pl.ds(i * BLK, BLK), :] - m_ref[...])


def _layer_body(h_ref, a0_ref, a1_ref, m_ref, w_ref, b_ref, g_ref, be_ref,
                hn_ref, mo_ref, p_ref):
    ph = pl.program_id(0)
    i = pl.program_id(1)

    @pl.when(ph == 0)
    def _():
        h = h_ref[...]
        acc = a0_ref[...] + a1_ref[...]
        has = jnp.max(acc, axis=1, keepdims=True) > 0.0
        agg = jnp.where(has,
                        m_ref[...] + jnp.log(jnp.maximum(acc, EPS_LSE)), 0.0)
        z = (jnp.dot(h, w_ref[:D, :], preferred_element_type=jnp.float32)
             + jnp.dot(agg, w_ref[D:, :], preferred_element_type=jnp.float32)
             + b_ref[...])
        mu = jnp.mean(z, axis=1, keepdims=True)
        zc = z - mu
        var = jnp.mean(zc * zc, axis=1, keepdims=True)
        zn = zc * lax.rsqrt(var + EPS_LN) * g_ref[...] + be_ref[...]
        hn = ALPHA * h + (1.0 - ALPHA) * jnp.maximum(zn, 0.0)
        hn_ref[pl.ds(i * BLK, BLK), :] = hn
        bm = jnp.max(hn, axis=0, keepdims=True)

        @pl.when(i == 0)
        def _():
            mo_ref[...] = bm

        @pl.when(i > 0)
        def _():
            mo_ref[...] = jnp.maximum(mo_ref[...], bm)

    @pl.when(ph == 1)
    def _():
        p_ref[...] = jnp.exp(hn_ref[pl.ds(i * BLK, BLK), :] - mo_ref[...])


_in_spec = pl.BlockSpec((BLK, D), lambda p, i: (i * (1 - p), 0))
_w_spec = lambda r: pl.BlockSpec((r, D), lambda p, i: (0, 0))
_vec_spec = pl.BlockSpec((1, D), lambda p, i: (0, 0))
_resident_spec = pl.BlockSpec((N, D), lambda p, i: (0, 0))
_p_spec = pl.BlockSpec((BLK, D), lambda p, i: (i * p, 0))

_proj = pl.pallas_call(
    _proj_body,
    grid=(2, NBLK),
    in_specs=[_in_spec, _w_spec(D), _vec_spec],
    out_specs=[_resident_spec, _vec_spec, _p_spec],
    out_shape=[jax.ShapeDtypeStruct((N, D), jnp.float32),
               jax.ShapeDtypeStruct((1, D), jnp.float32),
               jax.ShapeDtypeStruct((N, D), jnp.float32)],
)

_layer = pl.pallas_call(
    _layer_body,
    grid=(2, NBLK),
    in_specs=[_in_spec, _in_spec, _in_spec, _vec_spec, _w_spec(2 * D),
              _vec_spec, _vec_spec, _vec_spec],
    out_specs=[_resident_spec, _vec_spec, _p_spec],
    out_shape=[jax.ShapeDtypeStruct((N, D), jnp.float32),
               jax.ShapeDtypeStruct((1, D), jnp.float32),
               jax.ShapeDtypeStruct((N, D), jnp.float32)],
)


def _last_body(h_ref, a0_ref, a1_ref, m_ref, w_ref, b_ref, g_ref, be_ref,
               hn_ref):
    h = h_ref[...]
    acc = a0_ref[...] + a1_ref[...]
    has = jnp.max(acc, axis=1, keepdims=True) > 0.0
    agg = jnp.where(has,
                    m_ref[...] + jnp.log(jnp.maximum(acc, EPS_LSE)), 0.0)
    z = (jnp.dot(h, w_ref[:D, :], preferred_element_type=jnp.float32)
         + jnp.dot(agg, w_ref[D:, :], preferred_element_type=jnp.float32)
         + b_ref[...])
    mu = jnp.mean(z, axis=1, keepdims=True)
    zc = z - mu
    var = jnp.mean(zc * zc, axis=1, keepdims=True)
    zn = zc * lax.rsqrt(var + EPS_LN) * g_ref[...] + be_ref[...]
    hn_ref[...] = ALPHA * h + (1.0 - ALPHA) * jnp.maximum(zn, 0.0)


_blk_spec = pl.BlockSpec((BLK, D), lambda i: (i, 0))

_last = pl.pallas_call(
    _last_body,
    grid=(NBLK,),
    in_specs=[_blk_spec, _blk_spec, _blk_spec,
              pl.BlockSpec((1, D), lambda i: (0, 0)),
              pl.BlockSpec((2 * D, D), lambda i: (0, 0)),
              pl.BlockSpec((1, D), lambda i: (0, 0)),
              pl.BlockSpec((1, D), lambda i: (0, 0)),
              pl.BlockSpec((1, D), lambda i: (0, 0))],
    out_specs=_blk_spec,
    out_shape=jax.ShapeDtypeStruct((N, D), jnp.float32),
)


def _scatter_body(p_hbm, src_hbm, dst_hbm, out_hbm,
                  sbuf0, sbuf1, sbuf2, dbuf0, dbuf1, dbuf2,
                  rows0, rows1, rows2, acc,
                  sg0, sg1, sg2, sd0, sd1, sd2, ss0, ss1, ss2,
                  sr0, sr1, sr2):
    cid = lax.axis_index("c")
    sid = lax.axis_index("s")
    wid = sid * NC + cid
    sbuf = (sbuf0, sbuf1, sbuf2)
    dbuf = (dbuf0, dbuf1, dbuf2)
    rows = (rows0, rows1, rows2)
    sg = (sg0, sg1, sg2)
    sd = (sd0, sd1, sd2)
    ss = (ss0, ss1, ss2)
    sr = (sr0, sr1, sr2)

    pltpu.async_copy(src_hbm.at[wid, 0], sbuf0, sr0)
    pltpu.async_copy(src_hbm.at[wid, 1], sbuf1, sr1)
    pltpu.async_copy(dst_hbm.at[wid, 0], dbuf0, sd0)
    pltpu.async_copy(dst_hbm.at[wid, 1], dbuf1, sd1)
    pltpu.make_async_copy(src_hbm.at[wid, 0], sbuf0, sr0).wait()
    pltpu.async_copy(p_hbm.at[sbuf0], rows0, sg0)
    pltpu.make_async_copy(src_hbm.at[wid, 1], sbuf1, sr1).wait()
    pltpu.async_copy(p_hbm.at[sbuf1], rows1, sg1)
    pltpu.async_copy(src_hbm.at[wid, 2], sbuf2, sr2)

    zv = jnp.zeros((16,), jnp.float32)

    def zfill(r, _):
        for c in range(D // 16):
            rows2[r, pl.ds(c * 16, 16)] = zv
        return 0

    lax.fori_loop(0, CH, zfill, 0)

    def zcopy(j, _):
        pltpu.async_copy(rows2, acc.at[pl.ds(sid * RPT + j * CH, CH)], ss0)
        return 0

    lax.fori_loop(0, RPT // CH, zcopy, 0)

    zrem = RPT - (RPT // CH) * CH
    pltpu.async_copy(rows2.at[pl.ds(0, zrem)],
                     acc.at[pl.ds(sid * RPT + (RPT // CH) * CH, zrem)], ss0)

    @pl.when(sid == NS - 1)
    def _():
        pltpu.async_copy(rows2.at[pl.ds(0, TAIL)],
                         acc.at[pl.ds(NS * RPT, TAIL)], ss0)

    def zdrain(j, _):
        pltpu.make_async_copy(rows2, acc.at[pl.ds(sid * RPT + j * CH, CH)],
                              ss0).wait()
        return 0

    lax.fori_loop(0, RPT // CH, zdrain, 0)
    pltpu.make_async_copy(rows2.at[pl.ds(0, zrem)],
                          acc.at[pl.ds(sid * RPT, zrem)], ss0).wait()

    @pl.when(sid == NS - 1)
    def _():
        pltpu.make_async_copy(rows2.at[pl.ds(0, TAIL)],
                              acc.at[pl.ds(NS * RPT, TAIL)], ss0).wait()

    plsc.subcore_barrier()

    def stage(i, b):
        bp = (b + 2) % 3
        i = jnp.int32(i)
        pltpu.make_async_copy(p_hbm.at[sbuf[b]], rows[b], sg[b]).wait()
        pltpu.make_async_copy(dst_hbm.at[wid, i], dbuf[b], sd[b]).wait()
        pltpu.async_copy(rows[b], acc.at[dbuf[b]], ss[b], add=True)

        @pl.when(i >= 1)
        def _():
            pltpu.make_async_copy(rows[bp], acc.at[dbuf[bp]], ss[bp]).wait()

        @pl.when(i + 2 < NCHUNK)
        def _():
            pltpu.async_copy(dst_hbm.at[wid, i + 2], dbuf[bp], sd[bp])
            pltpu.make_async_copy(src_hbm.at[wid, i + 2], sbuf[bp],
                                  sr[bp]).wait()
            pltpu.async_copy(p_hbm.at[sbuf[bp]], rows[bp], sg[bp])

        @pl.when(i + 3 < NCHUNK)
        def _():
            pltpu.async_copy(src_hbm.at[wid, i + 3], sbuf[b], sr[b])

    def triple(j, _):
        stage(3 * j, 0)
        stage(3 * j + 1, 1)
        stage(3 * j + 2, 2)
        return 0

    nt = NCHUNK // 3
    lax.fori_loop(0, nt, triple, 0)
    for k in range(NCHUNK - 3 * nt):
        stage(3 * nt + k, k)
    lb = (NCHUNK - 1) % 3
    pltpu.make_async_copy(rows[lb], acc.at[dbuf[lb]], ss[lb]).wait()

    plsc.subcore_barrier()

    r0 = sid * RPT
    pltpu.sync_copy(acc.at[pl.ds(r0, RPT)], out_hbm.at[cid, pl.ds(r0, RPT)])

    @pl.when(sid == NS - 1)
    def _():
        pltpu.sync_copy(acc.at[pl.ds(NS * RPT, TAIL)],
                        out_hbm.at[cid, pl.ds(NS * RPT, TAIL)])


@functools.cache
def _make_scatter():
    return pl.kernel(
        _scatter_body,
        out_type=jax.ShapeDtypeStruct((NC, N, D), jnp.float32),
        mesh=plsc.VectorSubcoreMesh(core_axis_name="c", subcore_axis_name="s",
                                    num_cores=NC, num_subcores=NS),
        scratch_types=[
            pltpu.VMEM((CH,), jnp.int32),
            pltpu.VMEM((CH,), jnp.int32),
            pltpu.VMEM((CH,), jnp.int32),
            pltpu.VMEM((CH,), jnp.int32),
            pltpu.VMEM((CH,), jnp.int32),
            pltpu.VMEM((CH,), jnp.int32),
            pltpu.VMEM((CH, D), jnp.float32),
            pltpu.VMEM((CH, D), jnp.float32),
            pltpu.VMEM((CH, D), jnp.float32),
            pltpu.VMEM_SHARED((N, D), jnp.float32),
        ] + [pltpu.SemaphoreType.DMA] * 12,
    )


def kernel(x, edge_src, edge_dst, W_in, b_in, W1, b1, g1, be1,
           W2, b2, g2, be2, W3, b3, g3, be3):
    b_in = b_in.reshape(1, D)
    edge_src = edge_src.reshape(NW, NCHUNK, CH)
    edge_dst = edge_dst.reshape(NW, NCHUNK, CH)
    h, m, p = _proj(x, W_in, b_in)
    for (W, b, g, be) in ((W1, b1, g1, be1), (W2, b2, g2, be2)):
        parts = _make_scatter()(p, edge_src, edge_dst)
        h, m, p = _layer(h, parts[0], parts[1], m, W,
                         b.reshape(1, D), g.reshape(1, D), be.reshape(1, D))
    parts = _make_scatter()(p, edge_src, edge_dst)
    return _last(h, parts[0], parts[1], m, W3,
                 b3.reshape(1, D), g3.reshape(1, D), be3.reshape(1, D))


# speedup vs baseline: 1.1089x; 1.0031x over previous
import functools

import jax
import jax.numpy as jnp
from jax import lax
from jax.experimental import pallas as pl
from jax.experimental.pallas import tpu as pltpu
from jax.experimental.pallas import tpu_sc as plsc

N = 10000
E = 320000
D = 128
ALPHA = 0.5
EPS_LSE = 1e-30
EPS_LN = 1e-5

BLK = 5000
NBLK = N // BLK

NC = 2
NS = 16
NW = NC * NS
EPW = E // NW
CH = 125
NCHUNK = EPW // CH
RPT = 624
TAIL = N - NS * RPT


def _proj_body(x_ref, w_ref, b_ref, h_ref, m_ref, p_ref):
    ph = pl.program_id(0)
    i = pl.program_id(1)

    @pl.when(ph == 0)
    def _():
        h = jnp.dot(x_ref[...], w_ref[...], preferred_element_type=jnp.float32)
        h = h + b_ref[...]
        h_ref[pl.ds(i * BLK, BLK), :] = h
        bm = jnp.max(h, axis=0, keepdims=True)

        @pl.when(i == 0)
        def _():
            m_ref[...] = bm

        @pl.when(i > 0)
        def _():
            m_ref[...] = jnp.maximum(m_ref[...], bm)

    @pl.when(ph == 1)
    def _():
        p_ref[...] = jnp.exp(h_ref[pl.ds(i * BLK, BLK), :] - m_ref[...])


def _layer_body(h_ref, a0_ref, a1_ref, m_ref, w_ref, b_ref, g_ref, be_ref,
                hn_ref, mo_ref, p_ref):
    ph = pl.program_id(0)
    i = pl.program_id(1)

    @pl.when(ph == 0)
    def _():
        h = h_ref[...]
        acc = a0_ref[...] + a1_ref[...]
        has = jnp.max(acc, axis=1, keepdims=True) > 0.0
        agg = jnp.where(has,
                        m_ref[...] + jnp.log(jnp.maximum(acc, EPS_LSE)), 0.0)
        z = (jnp.dot(h, w_ref[:D, :], preferred_element_type=jnp.float32)
             + jnp.dot(agg, w_ref[D:, :], preferred_element_type=jnp.float32)
             + b_ref[...])
        mu = jnp.mean(z, axis=1, keepdims=True)
        zc = z - mu
        var = jnp.mean(zc * zc, axis=1, keepdims=True)
        zn = zc * lax.rsqrt(var + EPS_LN) * g_ref[...] + be_ref[...]
        hn = ALPHA * h + (1.0 - ALPHA) * jnp.maximum(zn, 0.0)
        hn_ref[pl.ds(i * BLK, BLK), :] = hn
        bm = jnp.max(hn, axis=0, keepdims=True)

        @pl.when(i == 0)
        def _():
            mo_ref[...] = bm

        @pl.when(i > 0)
        def _():
            mo_ref[...] = jnp.maximum(mo_ref[...], bm)

    @pl.when(ph == 1)
    def _():
        p_ref[...] = jnp.exp(hn_ref[pl.ds(i * BLK, BLK), :] - mo_ref[...])


_in_spec = pl.BlockSpec((BLK, D), lambda p, i: (i * (1 - p) + (NBLK - 1) * p, 0))
_w_spec = lambda r: pl.BlockSpec((r, D), lambda p, i: (0, 0))
_vec_spec = pl.BlockSpec((1, D), lambda p, i: (0, 0))
_resident_spec = pl.BlockSpec((N, D), lambda p, i: (0, 0))
_p_spec = pl.BlockSpec((BLK, D), lambda p, i: (i * p, 0))

_proj = pl.pallas_call(
    _proj_body,
    grid=(2, NBLK),
    in_specs=[_in_spec, _w_spec(D), _vec_spec],
    out_specs=[_resident_spec, _vec_spec, _p_spec],
    out_shape=[jax.ShapeDtypeStruct((N, D), jnp.float32),
               jax.ShapeDtypeStruct((1, D), jnp.float32),
               jax.ShapeDtypeStruct((N, D), jnp.float32)],
)

_layer = pl.pallas_call(
    _layer_body,
    grid=(2, NBLK),
    in_specs=[_in_spec, _in_spec, _in_spec, _vec_spec, _w_spec(2 * D),
              _vec_spec, _vec_spec, _vec_spec],
    out_specs=[_resident_spec, _vec_spec, _p_spec],
    out_shape=[jax.ShapeDtypeStruct((N, D), jnp.float32),
               jax.ShapeDtypeStruct((1, D), jnp.float32),
               jax.ShapeDtypeStruct((N, D), jnp.float32)],
)


def _last_body(h_ref, a0_ref, a1_ref, m_ref, w_ref, b_ref, g_ref, be_ref,
               hn_ref):
    h = h_ref[...]
    acc = a0_ref[...] + a1_ref[...]
    has = jnp.max(acc, axis=1, keepdims=True) > 0.0
    agg = jnp.where(has,
                    m_ref[...] + jnp.log(jnp.maximum(acc, EPS_LSE)), 0.0)
    z = (jnp.dot(h, w_ref[:D, :], preferred_element_type=jnp.float32)
         + jnp.dot(agg, w_ref[D:, :], preferred_element_type=jnp.float32)
         + b_ref[...])
    mu = jnp.mean(z, axis=1, keepdims=True)
    zc = z - mu
    var = jnp.mean(zc * zc, axis=1, keepdims=True)
    zn = zc * lax.rsqrt(var + EPS_LN) * g_ref[...] + be_ref[...]
    hn_ref[...] = ALPHA * h + (1.0 - ALPHA) * jnp.maximum(zn, 0.0)


_blk_spec = pl.BlockSpec((BLK, D), lambda i: (i, 0))

_last = pl.pallas_call(
    _last_body,
    grid=(NBLK,),
    in_specs=[_blk_spec, _blk_spec, _blk_spec,
              pl.BlockSpec((1, D), lambda i: (0, 0)),
              pl.BlockSpec((2 * D, D), lambda i: (0, 0)),
              pl.BlockSpec((1, D), lambda i: (0, 0)),
              pl.BlockSpec((1, D), lambda i: (0, 0)),
              pl.BlockSpec((1, D), lambda i: (0, 0))],
    out_specs=_blk_spec,
    out_shape=jax.ShapeDtypeStruct((N, D), jnp.float32),
)


def _scatter_body(p_hbm, src_hbm, dst_hbm, out_hbm,
                  sbuf0, sbuf1, sbuf2, dbuf0, dbuf1, dbuf2,
                  rows0, rows1, rows2, acc,
                  sg0, sg1, sg2, sd0, sd1, sd2, ss0, ss1, ss2,
                  sr0, sr1, sr2):
    cid = lax.axis_index("c")
    sid = lax.axis_index("s")
    wid = sid * NC + cid
    sbuf = (sbuf0, sbuf1, sbuf2)
    dbuf = (dbuf0, dbuf1, dbuf2)
    rows = (rows0, rows1, rows2)
    sg = (sg0, sg1, sg2)
    sd = (sd0, sd1, sd2)
    ss = (ss0, ss1, ss2)
    sr = (sr0, sr1, sr2)

    pltpu.async_copy(src_hbm.at[wid, 0], sbuf0, sr0)
    pltpu.async_copy(src_hbm.at[wid, 1], sbuf1, sr1)
    pltpu.async_copy(dst_hbm.at[wid, 0], dbuf0, sd0)
    pltpu.async_copy(dst_hbm.at[wid, 1], dbuf1, sd1)
    pltpu.make_async_copy(src_hbm.at[wid, 0], sbuf0, sr0).wait()
    pltpu.async_copy(p_hbm.at[sbuf0], rows0, sg0)
    pltpu.make_async_copy(src_hbm.at[wid, 1], sbuf1, sr1).wait()
    pltpu.async_copy(p_hbm.at[sbuf1], rows1, sg1)
    pltpu.async_copy(src_hbm.at[wid, 2], sbuf2, sr2)

    zv = jnp.zeros((16,), jnp.float32)

    def zfill(r, _):
        for c in range(D // 16):
            rows2[r, pl.ds(c * 16, 16)] = zv
        return 0

    lax.fori_loop(0, CH, zfill, 0)

    def zcopy(j, _):
        pltpu.async_copy(rows2, acc.at[pl.ds(sid * RPT + j * CH, CH)], ss0)
        return 0

    lax.fori_loop(0, RPT // CH, zcopy, 0)

    zrem = RPT - (RPT // CH) * CH
    pltpu.async_copy(rows2.at[pl.ds(0, zrem)],
                     acc.at[pl.ds(sid * RPT + (RPT // CH) * CH, zrem)], ss0)

    @pl.when(sid == NS - 1)
    def _():
        pltpu.async_copy(rows2.at[pl.ds(0, TAIL)],
                         acc.at[pl.ds(NS * RPT, TAIL)], ss0)

    def zdrain(j, _):
        pltpu.make_async_copy(rows2, acc.at[pl.ds(sid * RPT + j * CH, CH)],
                              ss0).wait()
        return 0

    lax.fori_loop(0, RPT // CH, zdrain, 0)
    pltpu.make_async_copy(rows2.at[pl.ds(0, zrem)],
                          acc.at[pl.ds(sid * RPT, zrem)], ss0).wait()

    @pl.when(sid == NS - 1)
    def _():
        pltpu.make_async_copy(rows2.at[pl.ds(0, TAIL)],
                              acc.at[pl.ds(NS * RPT, TAIL)], ss0).wait()

    plsc.subcore_barrier()

    def stage(i, b):
        bp = (b + 2) % 3
        i = jnp.int32(i)
        pltpu.make_async_copy(p_hbm.at[sbuf[b]], rows[b], sg[b]).wait()
        pltpu.make_async_copy(dst_hbm.at[wid, i], dbuf[b], sd[b]).wait()
        pltpu.async_copy(rows[b], acc.at[dbuf[b]], ss[b], add=True)

        @pl.when(i >= 1)
        def _():
            pltpu.make_async_copy(rows[bp], acc.at[dbuf[bp]], ss[bp]).wait()

        @pl.when(i + 2 < NCHUNK)
        def _():
            pltpu.async_copy(dst_hbm.at[wid, i + 2], dbuf[bp], sd[bp])
            pltpu.make_async_copy(src_hbm.at[wid, i + 2], sbuf[bp],
                                  sr[bp]).wait()
            pltpu.async_copy(p_hbm.at[sbuf[bp]], rows[bp], sg[bp])

        @pl.when(i + 3 < NCHUNK)
        def _():
            pltpu.async_copy(src_hbm.at[wid, i + 3], sbuf[b], sr[b])

    def triple(j, _):
        stage(3 * j, 0)
        stage(3 * j + 1, 1)
        stage(3 * j + 2, 2)
        return 0

    nt = NCHUNK // 3
    lax.fori_loop(0, nt, triple, 0)
    for k in range(NCHUNK - 3 * nt):
        stage(3 * nt + k, k)
    lb = (NCHUNK - 1) % 3
    pltpu.make_async_copy(rows[lb], acc.at[dbuf[lb]], ss[lb]).wait()

    plsc.subcore_barrier()

    r0 = sid * RPT
    pltpu.sync_copy(acc.at[pl.ds(r0, RPT)], out_hbm.at[cid, pl.ds(r0, RPT)])

    @pl.when(sid == NS - 1)
    def _():
        pltpu.sync_copy(acc.at[pl.ds(NS * RPT, TAIL)],
                        out_hbm.at[cid, pl.ds(NS * RPT, TAIL)])


@functools.cache
def _make_scatter():
    return pl.kernel(
        _scatter_body,
        out_type=jax.ShapeDtypeStruct((NC, N, D), jnp.float32),
        mesh=plsc.VectorSubcoreMesh(core_axis_name="c", subcore_axis_name="s",
                                    num_cores=NC, num_subcores=NS),
        scratch_types=[
            pltpu.VMEM((CH,), jnp.int32),
            pltpu.VMEM((CH,), jnp.int32),
            pltpu.VMEM((CH,), jnp.int32),
            pltpu.VMEM((CH,), jnp.int32),
            pltpu.VMEM((CH,), jnp.int32),
            pltpu.VMEM((CH,), jnp.int32),
            pltpu.VMEM((CH, D), jnp.float32),
            pltpu.VMEM((CH, D), jnp.float32),
            pltpu.VMEM((CH, D), jnp.float32),
            pltpu.VMEM_SHARED((N, D), jnp.float32),
        ] + [pltpu.SemaphoreType.DMA] * 12,
    )


def kernel(x, edge_src, edge_dst, W_in, b_in, W1, b1, g1, be1,
           W2, b2, g2, be2, W3, b3, g3, be3):
    b_in = b_in.reshape(1, D)
    edge_src = edge_src.reshape(NW, NCHUNK, CH)
    edge_dst = edge_dst.reshape(NW, NCHUNK, CH)
    h, m, p = _proj(x, W_in, b_in)
    for (W, b, g, be) in ((W1, b1, g1, be1), (W2, b2, g2, be2)):
        parts = _make_scatter()(p, edge_src, edge_dst)
        h, m, p = _layer(h, parts[0], parts[1], m, W,
                         b.reshape(1, D), g.reshape(1, D), be.reshape(1, D))
    parts = _make_scatter()(p, edge_src, edge_dst)
    return _last(h, parts[0], parts[1], m, W3,
                 b3.reshape(1, D), g3.reshape(1, D), be3.reshape(1, D))
